# baseline (speedup 1.0000x reference)
import functools

import jax
import jax.numpy as jnp
from jax import lax
from jax.experimental import pallas as pl
from jax.experimental.pallas import tpu as pltpu

KNN = 20
EPS = 1e-5


def _lrelu(x):
    return jnp.where(x >= 0, x, 0.2 * x)


def _apply_bn_kernel(maxu_ref, v_ref, a_ref, c_ref, o_ref):
    y = (maxu_ref[...] + v_ref[...]) * a_ref[0] + c_ref[0]
    o_ref[...] = jnp.where(y >= 0, y, 0.2 * y)


def _apply_bn(maxu, vT, a, c):
    B, N, O = maxu.shape
    a2 = jnp.broadcast_to(a[None, :], (8, O))
    c2 = jnp.broadcast_to(c[None, :], (8, O))
    grid = (B,)
    return pl.pallas_call(
        _apply_bn_kernel,
        grid=grid,
        in_specs=[
            pl.BlockSpec((1, N, O), lambda b: (b, 0, 0)),
            pl.BlockSpec((1, N, O), lambda b: (b, 0, 0)),
            pl.BlockSpec((8, O), lambda b: (0, 0)),
            pl.BlockSpec((8, O), lambda b: (0, 0)),
        ],
        out_specs=pl.BlockSpec((1, N, O), lambda b: (b, 0, 0)),
        out_shape=jax.ShapeDtypeStruct((B, N, O), jnp.float32),
    )(maxu, vT, a2, c2)


def _edge_conv(xT, W, g, b):
    B, N, C = xT.shape
    xx = jnp.sum(xT * xT, axis=-1)
    inner = jnp.einsum('bnc,bmc->bnm', xT, xT)
    pd = 2.0 * inner - xx[:, :, None] - xx[:, None, :]
    idx = lax.top_k(pd, KNN)[1]

    Wd = W[:, :C]
    Wc = W[:, C:]
    uT = jnp.einsum('bnc,oc->bno', xT, Wd)
    vT = jnp.einsum('bnc,oc->bno', xT, Wc - Wd)

    O = W.shape[0]
    uflat = uT.reshape(B * N, O)
    gidx = (idx + (jnp.arange(B) * N)[:, None, None]).reshape(-1)
    ug = uflat[gidx].reshape(B, N, KNN, O)
    maxu = ug.max(axis=2)
    s1 = ug.sum(axis=2)
    s2 = (ug * ug).sum(axis=2)

    cnt = B * N * KNN
    S1 = s1.sum(axis=(0, 1))
    SV = vT.sum(axis=(0, 1))
    mean = (S1 + KNN * SV) / cnt
    Ey2 = (s2.sum(axis=(0, 1)) + 2.0 * (vT * s1).sum(axis=(0, 1))
           + KNN * (vT * vT).sum(axis=(0, 1))) / cnt
    var = Ey2 - mean * mean
    a = g / jnp.sqrt(var + EPS)
    c = b - mean * a
    return _apply_bn(maxu, vT, a, c)


def _level_feat(x5T, Wf, bf, gf, bbf):
    f = jnp.einsum('bnc,oc->bno', x5T, Wf) + bf[None, None, :]
    mean = jnp.mean(f, axis=1, keepdims=True)
    var = jnp.mean((f - mean) ** 2, axis=1, keepdims=True)
    return gf[None, None, :] * (f - mean) / jnp.sqrt(var + EPS) + bbf[None, None, :]


def kernel(pt_coord, feats, W1, g1, b1, W2, g2, b2, W3, g3, b3, W4, g4, b4,
           W5, g5, b5, Wf0, bf0, gf0, bbf0, Wf1, bf1, gf1, bbf1,
           Wf2, bf2, gf2, bbf2, Wf3, bf3, gf3, bbf3, Wsem, bsem):
    B, N = pt_coord.shape[0], pt_coord.shape[1]
    x0 = jnp.concatenate([pt_coord, feats[:, :, 3:]], axis=2)

    x1 = _edge_conv(x0, W1, g1, b1)
    x2 = _edge_conv(x1, W2, g2, b2)
    x3 = _edge_conv(x2, W3, g3, b3)
    x4 = _edge_conv(x3, W4, g4, b4)

    xc = jnp.concatenate([x1, x2, x3, x4], axis=2)
    y5 = jnp.einsum('bnc,oc->bno', xc, W5)
    m5 = jnp.mean(y5, axis=(0, 1))
    v5 = jnp.mean((y5 - m5) ** 2, axis=(0, 1))
    x5 = _lrelu(g5 * (y5 - m5) / jnp.sqrt(v5 + EPS) + b5)

    ms = [
        _level_feat(x5, Wf0, bf0, gf0, bbf0),
        _level_feat(x5, Wf1, bf1, gf1, bbf1),
        _level_feat(x5, Wf2, bf2, gf2, bbf2),
        _level_feat(x5, Wf3, bf3, gf3, bbf3),
    ]
    sem = jnp.einsum('bnc,oc->bno', ms[3], Wsem) + bsem[None, None, :]

    masks = jnp.zeros((B, N), dtype=bool)
    return (tuple(ms), (pt_coord,) * 4, (masks,) * 4, sem)


# speedup vs baseline: 6.2877x; 6.2877x over previous
import functools

import jax
import jax.numpy as jnp
from jax import lax
from jax.experimental import pallas as pl
from jax.experimental.pallas import tpu as pltpu
from jax.experimental.pallas import tpu_sc as plsc

KNN = 20
EPS = 1e-5
NEG = -3.0e38


def _knn_body(xf_ref, xb_ref, idx_ref, *, N, R, k):
    b = pl.program_id(0)
    xf = xf_ref[0]
    xblk = xb_ref[0]
    xfb = xf.astype(jnp.bfloat16)
    xbb = xblk.astype(jnp.bfloat16)
    inner = lax.dot_general(xbb, xfb, (((1,), (1,)), ((), ())),
                            preferred_element_type=jnp.float32)
    xxf = jnp.sum(xf * xf, axis=1)
    xxb = jnp.sum(xblk * xblk, axis=1)
    pd = 2.0 * inner - xxb[:, None] - xxf[None, :]
    iota = lax.broadcasted_iota(jnp.int32, (R, N), 1)
    cols = []
    for _ in range(k):
        m = jnp.max(pd, axis=1, keepdims=True)
        eq = pd == m
        cand = jnp.where(eq, iota, N)
        sel = jnp.min(cand, axis=1, keepdims=True)
        cols.append(sel)
        pd = jnp.where(iota == sel, NEG, pd)
    idx_ref[...] = jnp.concatenate(cols, axis=1) + b * N


def _knn_topk(xT, R=512):
    B, N, C = xT.shape
    NB = N // R
    return pl.pallas_call(
        functools.partial(_knn_body, N=N, R=R, k=KNN),
        grid=(B, NB),
        in_specs=[
            pl.BlockSpec((1, N, C), lambda b, nb: (b, 0, 0)),
            pl.BlockSpec((1, R, C), lambda b, nb: (b, nb, 0)),
        ],
        out_specs=pl.BlockSpec((R, KNN), lambda b, nb: (b * NB + nb, 0)),
        out_shape=jax.ShapeDtypeStruct((B * N, KNN), jnp.int32),
    )(xT, xT)


def _sc_gather(table, gidx):
    M, C = table.shape
    E = gidx.shape[0]
    info = plsc.get_sparse_core_info()
    NC, NS = info.num_cores, info.num_subcores
    NW = NC * NS
    per_w = E // NW
    CH = 80
    n_ch = per_w // CH
    assert per_w % CH == 0 and E % NW == 0
    mesh = plsc.VectorSubcoreMesh(core_axis_name="c", subcore_axis_name="s")

    @functools.partial(
        pl.kernel, mesh=mesh,
        out_type=jax.ShapeDtypeStruct((E, C), jnp.float32),
        scratch_types=[
            pltpu.VMEM((CH,), jnp.int32),
            pltpu.VMEM((CH, C), jnp.float32),
            pltpu.SemaphoreType.DMA,
        ],
    )
    def gather_k(table_hbm, idx_hbm, out_hbm, idx_v, rows_v, sem):
        wid = lax.axis_index("s") * NC + lax.axis_index("c")
        base0 = wid * per_w

        def body(ci, carry):
            base = base0 + ci * CH
            pltpu.sync_copy(idx_hbm.at[pl.ds(base, CH)], idx_v)
            pltpu.async_copy(table_hbm.at[idx_v], rows_v, sem).wait()
            pltpu.sync_copy(rows_v, out_hbm.at[pl.ds(base, CH)])
            return carry

        lax.fori_loop(0, n_ch, body, 0)

    return gather_k(table, gidx)


def _edge_body(xg_ref, xc_ref, w_ref, maxy_ref, stats_ref, *, R, k, O, creal):
    b = pl.program_id(0)
    nb = pl.program_id(1)
    first = (b == 0) & (nb == 0)
    xg = xg_ref[0, 0]
    xc = xc_ref[0]
    C = xc.shape[-1]
    xg3 = xg.reshape(R, k, C)[:, :, :creal]
    xcr = xc[:, :creal]
    diff = (xg3 - xcr[:, None, :]).astype(jnp.bfloat16)
    ctr = jnp.broadcast_to(xcr[:, None, :], (R, k, creal)).astype(jnp.bfloat16)
    fe = jnp.concatenate([diff, ctr], axis=2).reshape(R * k, 2 * creal)
    y = lax.dot_general(fe, w_ref[...], (((1,), (0,)), ((), ())),
                        preferred_element_type=jnp.float32)
    maxy_ref[0] = jnp.max(y.reshape(R, k, O), axis=1)

    @pl.when(first)
    def _():
        stats_ref[...] = jnp.zeros_like(stats_ref)

    stats_ref[0, :] += jnp.sum(y, axis=0)
    stats_ref[1, :] += jnp.sum(y * y, axis=0)


def _edge_conv_mm(xg, xT, wT, R=128):
    B, N, C = xT.shape
    O = wT.shape[1]
    creal = wT.shape[0] // 2
    NB = N // R
    xg4 = xg.reshape(B, NB, R * KNN, C)
    return pl.pallas_call(
        functools.partial(_edge_body, R=R, k=KNN, O=O, creal=creal),
        grid=(B, NB),
        in_specs=[
            pl.BlockSpec((1, 1, R * KNN, C), lambda b, nb: (b, nb, 0, 0)),
            pl.BlockSpec((1, R, C), lambda b, nb: (b, nb, 0)),
            pl.BlockSpec((2 * creal, O), lambda b, nb: (0, 0)),
        ],
        out_specs=[
            pl.BlockSpec((1, R, O), lambda b, nb: (b, nb, 0)),
            pl.BlockSpec((8, O), lambda b, nb: (0, 0)),
        ],
        out_shape=[
            jax.ShapeDtypeStruct((B, N, O), jnp.float32),
            jax.ShapeDtypeStruct((8, O), jnp.float32),
        ],
    )(xg4, xT, wT)


def _apply_body(my_ref, st_ref, g_ref, b_ref, o_ref, *, cnt, opad):
    sy = st_ref[0]
    sy2 = st_ref[1]
    mean = sy / cnt
    var = sy2 / cnt - mean * mean
    y = g_ref[0] * (my_ref[0] - mean) / jnp.sqrt(var + EPS) + b_ref[0]
    y = jnp.where(y >= 0, y, 0.2 * y)
    if opad > y.shape[1]:
        y = jnp.concatenate(
            [y, jnp.zeros((y.shape[0], opad - y.shape[1]), y.dtype)], axis=1)
    o_ref[0] = y


def _apply_bn(maxy, stats, g, b, opad):
    B, N, O = maxy.shape
    g2 = jnp.broadcast_to(g[None, :], (8, O))
    b2 = jnp.broadcast_to(b[None, :], (8, O))
    return pl.pallas_call(
        functools.partial(_apply_body, cnt=float(B * N * KNN), opad=opad),
        grid=(B,),
        in_specs=[
            pl.BlockSpec((1, N, O), lambda b_: (b_, 0, 0)),
            pl.BlockSpec((8, O), lambda b_: (0, 0)),
            pl.BlockSpec((8, O), lambda b_: (0, 0)),
            pl.BlockSpec((8, O), lambda b_: (0, 0)),
        ],
        out_specs=pl.BlockSpec((1, N, opad), lambda b_: (b_, 0, 0)),
        out_shape=jax.ShapeDtypeStruct((B, N, opad), jnp.float32),
    )(maxy, stats, g2, b2)


def _edge_conv(xT, W, g, b):
    B, N, C = xT.shape
    creal = W.shape[1] // 2
    O = W.shape[0]
    wT = W.T.astype(jnp.bfloat16)
    idx = _knn_topk(xT)
    xg = _sc_gather(xT.reshape(B * N, C), idx.reshape(-1))
    maxy, stats = _edge_conv_mm(xg, xT, wT)
    return _apply_bn(maxy, stats, g, b, max(O, 128))


def _head1_body(x1_ref, x2_ref, x3_ref, x4_ref, w_ref, y5_ref, st_ref):
    first = (pl.program_id(0) == 0) & (pl.program_id(1) == 0)
    xcb = jnp.concatenate(
        [x1_ref[0][:, :64], x2_ref[0][:, :64], x3_ref[0], x4_ref[0]], axis=1
    ).astype(jnp.bfloat16)
    y5 = lax.dot_general(xcb, w_ref[...], (((1,), (0,)), ((), ())),
                         preferred_element_type=jnp.float32)

    @pl.when(first)
    def _():
        st_ref[...] = jnp.zeros_like(st_ref)

    y5_ref[0] = y5
    st_ref[0, :] += jnp.sum(y5, axis=0)
    st_ref[1, :] += jnp.sum(y5 * y5, axis=0)


def _head1(x1, x2, x3, x4, W5, R=512):
    B, N, _ = x1.shape
    NB = N // R
    w5T = W5.T.astype(jnp.bfloat16)
    return pl.pallas_call(
        _head1_body,
        grid=(B, NB),
        in_specs=[
            pl.BlockSpec((1, R, x.shape[2]), lambda b, nb: (b, nb, 0))
            for x in (x1, x2, x3, x4)
        ] + [pl.BlockSpec(w5T.shape, lambda b, nb: (0, 0))],
        out_specs=[
            pl.BlockSpec((1, R, 512), lambda b, nb: (b, nb, 0)),
            pl.BlockSpec((8, 512), lambda b, nb: (0, 0)),
        ],
        out_shape=[
            jax.ShapeDtypeStruct((B, N, 512), jnp.float32),
            jax.ShapeDtypeStruct((8, 512), jnp.float32),
        ],
    )(x1, x2, x3, x4, w5T)


def _head2_body(y5_ref, st_ref, g5_ref, b5_ref,
                wf0_ref, p0_ref, wf1_ref, p1_ref,
                wf2_ref, p2_ref, wf3_ref, p3_ref,
                wsem_ref, bsem_ref,
                ms0_ref, ms1_ref, ms2_ref, ms3_ref, sem_ref, *, cnt5):
    y5 = y5_ref[0]
    mean5 = st_ref[0] / cnt5
    var5 = st_ref[1] / cnt5 - mean5 * mean5
    x5 = g5_ref[0] * (y5 - mean5) / jnp.sqrt(var5 + EPS) + b5_ref[0]
    x5 = jnp.where(x5 >= 0, x5, 0.2 * x5)
    x5b = x5.astype(jnp.bfloat16)

    def level(wf_ref, p_ref):
        f = lax.dot_general(x5b, wf_ref[...], (((1,), (0,)), ((), ())),
                            preferred_element_type=jnp.float32) + p_ref[0]
        mean = jnp.mean(f, axis=0, keepdims=True)
        var = jnp.mean((f - mean) ** 2, axis=0, keepdims=True)
        return p_ref[2] * (f - mean) / jnp.sqrt(var + EPS) + p_ref[3]

    ms0 = level(wf0_ref, p0_ref)
    ms1 = level(wf1_ref, p1_ref)
    ms2 = level(wf2_ref, p2_ref)
    ms3 = level(wf3_ref, p3_ref)
    ms0_ref[0] = ms0
    ms1_ref[0] = ms1
    ms2_ref[0] = ms2
    ms3_ref[0] = ms3
    sem_ref[0] = lax.dot_general(
        ms3.astype(jnp.bfloat16), wsem_ref[...], (((1,), (0,)), ((), ())),
        preferred_element_type=jnp.float32) + bsem_ref[0]


def _head2(y5, st5, g5, b5, wfs, pks, wsem_p, bsem_p):
    B, N, _ = y5.shape
    cs = [w.shape[1] for w in wfs]
    specs = [
        pl.BlockSpec((1, N, 512), lambda b: (b, 0, 0)),
        pl.BlockSpec((8, 512), lambda b: (0, 0)),
        pl.BlockSpec((8, 512), lambda b: (0, 0)),
        pl.BlockSpec((8, 512), lambda b: (0, 0)),
    ]
    args = [y5, st5, g5, b5]
    for w, p in zip(wfs, pks):
        specs.append(pl.BlockSpec(w.shape, lambda b: (0, 0)))
        specs.append(pl.BlockSpec(p.shape, lambda b: (0, 0)))
        args.append(w)
        args.append(p)
    specs.append(pl.BlockSpec(wsem_p.shape, lambda b: (0, 0)))
    specs.append(pl.BlockSpec((8, 128), lambda b: (0, 0)))
    args.append(wsem_p)
    args.append(bsem_p)
    return pl.pallas_call(
        functools.partial(_head2_body, cnt5=float(B * N)),
        grid=(B,),
        in_specs=specs,
        out_specs=[pl.BlockSpec((1, N, c), lambda b: (b, 0, 0)) for c in cs]
        + [pl.BlockSpec((1, N, 128), lambda b: (b, 0, 0))],
        out_shape=[jax.ShapeDtypeStruct((B, N, c), jnp.float32) for c in cs]
        + [jax.ShapeDtypeStruct((B, N, 128), jnp.float32)],
    )(*args)


def _bcast8(v, n):
    out = jnp.zeros((8, n), jnp.float32)
    return out.at[:, :v.shape[0]].set(jnp.broadcast_to(v[None, :], (8, v.shape[0])))


def kernel(pt_coord, feats, W1, g1, b1, W2, g2, b2, W3, g3, b3, W4, g4, b4,
           W5, g5, b5, Wf0, bf0, gf0, bbf0, Wf1, bf1, gf1, bbf1,
           Wf2, bf2, gf2, bbf2, Wf3, bf3, gf3, bbf3, Wsem, bsem):
    B, N = pt_coord.shape[0], pt_coord.shape[1]
    x0 = jnp.concatenate([pt_coord, feats[:, :, 3:]], axis=2)
    x0 = jnp.pad(x0, ((0, 0), (0, 0), (0, 122)))

    x1 = _edge_conv(x0, W1, g1, b1)
    x2 = _edge_conv(x1, W2, g2, b2)
    x3 = _edge_conv(x2, W3, g3, b3)
    x4 = _edge_conv(x3, W4, g4, b4)

    y5, st5 = _head1(x1, x2, x3, x4, W5)

    wfs = [Wf0.T.astype(jnp.bfloat16), Wf1.T.astype(jnp.bfloat16),
           Wf2.T.astype(jnp.bfloat16), Wf3.T.astype(jnp.bfloat16)]
    pks = []
    for bf, gf, bbf in ((bf0, gf0, bbf0), (bf1, gf1, bbf1),
                        (bf2, gf2, bbf2), (bf3, gf3, bbf3)):
        c = bf.shape[0]
        pk = jnp.zeros((8, c), jnp.float32)
        pk = pk.at[0].set(bf).at[2].set(gf).at[3].set(bbf)
        pks.append(pk)
    wsem_p = jnp.zeros((256, 128), jnp.float32).at[:, :20].set(Wsem.T).astype(jnp.bfloat16)
    bsem_p = jnp.zeros((8, 128), jnp.float32).at[0, :20].set(bsem)

    ms0, ms1, ms2, ms3, sem_p = _head2(
        y5, st5, _bcast8(g5, 512), _bcast8(b5, 512), wfs, pks, wsem_p, bsem_p)
    sem = sem_p[:, :, :20]

    masks = jnp.zeros((B, N), dtype=bool)
    return ((ms0, ms1, ms2, ms3), (pt_coord,) * 4, (masks,) * 4, sem)
